# baseline (speedup 1.0000x reference)
import jax
import jax.numpy as jnp
import numpy as np
from jax.experimental import pallas as pl

T_STEPS = 32
T_CHUNK = 32


def _body(x_ref, o_ref):
    xb = x_ref[...]
    xc = jnp.clip(xb, 0.0, 1.0)
    t = (T_STEPS * (1.0 - xc)).astype(jnp.int32)
    t = jnp.clip(t, 0, T_STEPS - 1)
    C, H, W = xb.shape[1:]
    t_base = pl.program_id(1) * T_CHUNK
    tio = t_base + jax.lax.broadcasted_iota(
        jnp.int32, (1, T_CHUNK, C, H, W), 1
    )
    o_ref[...] = (tio == t[:, None]).astype(jnp.float32)


def kernel(x):
    B, C, H, W = x.shape
    out = pl.pallas_call(
        _body,
        grid=(B, T_STEPS // T_CHUNK),
        in_specs=[pl.BlockSpec((1, C, H, W), lambda b, tc: (b, 0, 0, 0))],
        out_specs=pl.BlockSpec(
            (1, T_CHUNK, C, H, W), lambda b, tc: (b, tc, 0, 0, 0)
        ),
        out_shape=jax.ShapeDtypeStruct((B, T_STEPS, C, H, W), jnp.float32),
    )(x)
    return out


# speedup vs baseline: 1.0148x; 1.0148x over previous
import jax
import jax.numpy as jnp
import numpy as np
from jax.experimental import pallas as pl

T_STEPS = 32
T_CHUNK = 16


def _body(x_ref, o_ref):
    xb = x_ref[...]
    xc = jnp.clip(xb, 0.0, 1.0)
    t = (T_STEPS * (1.0 - xc)).astype(jnp.int32)
    t = jnp.clip(t, 0, T_STEPS - 1)
    C, H, W = xb.shape[1:]
    t_base = pl.program_id(1) * T_CHUNK
    tio = t_base + jax.lax.broadcasted_iota(
        jnp.int32, (1, T_CHUNK, C, H, W), 1
    )
    o_ref[...] = (tio == t[:, None]).astype(jnp.float32)


def kernel(x):
    B, C, H, W = x.shape
    out = pl.pallas_call(
        _body,
        grid=(B, T_STEPS // T_CHUNK),
        in_specs=[pl.BlockSpec((1, C, H, W), lambda b, tc: (b, 0, 0, 0))],
        out_specs=pl.BlockSpec(
            (1, T_CHUNK, C, H, W), lambda b, tc: (b, tc, 0, 0, 0)
        ),
        out_shape=jax.ShapeDtypeStruct((B, T_STEPS, C, H, W), jnp.float32),
    )(x)
    return out
